# baseline (speedup 1.0000x reference)
import functools

import numpy as np
import jax
import jax.numpy as jnp
from jax import lax
from jax.experimental import pallas as pl
from jax.experimental.pallas import tpu as pltpu
from jax.experimental.pallas import tpu_sc as plsc

_N = 16384
_NUM_GROUPS = 16
_LANES = 16
_RECIP_BASE = _N + 1
_TABLE_LEN = 16416


def _const_table() -> np.ndarray:
    logs = np.log(np.arange(1, _N + 1, dtype=np.float64))
    t = np.zeros((_TABLE_LEN,), np.float64)
    t[1:_N + 1] = np.cumsum(logs)
    t[_RECIP_BASE:_RECIP_BASE + _NUM_GROUPS] = (
        1.0 / np.arange(1, _NUM_GROUPS + 1, dtype=np.float64))
    return t.astype(np.float32)


_LF_TABLE = _const_table()


@functools.partial(
    pl.kernel,
    mesh=plsc.VectorSubcoreMesh(core_axis_name="c", subcore_axis_name="s",
                                num_cores=1),
    out_type=jax.ShapeDtypeStruct((_LANES,), jnp.float32),
    compiler_params=pltpu.CompilerParams(needs_layout_passes=False),
    scratch_types=[
        pltpu.VMEM((_N,), jnp.int32),
        pltpu.VMEM((_TABLE_LEN,), jnp.float32),
        pltpu.VMEM((2 * _LANES,), jnp.int32),
        pltpu.VMEM((_LANES,), jnp.float32),
        pltpu.SemaphoreType.DMA,
        pltpu.SemaphoreType.DMA,
    ],
)
def _pl_listwise_loss(ids_hbm, lf_hbm, loss_hbm,
                      ids_v, lf_v, cnt2_v, outf_v, sem_i, sem_t):
    c = lax.axis_index("c")
    s = lax.axis_index("s")

    @pl.when((c == 0) & (s == 0))
    def _tile0():
        ids_copy = pltpu.async_copy(ids_hbm, ids_v, sem_i)
        lf_copy = pltpu.async_copy(lf_hbm, lf_v, sem_t)
        ids_copy.wait()

        lane = lax.iota(jnp.int32, _LANES)
        lo = jnp.zeros((_LANES,), jnp.int32)
        hi = jnp.full((_LANES,), _N, jnp.int32)
        for _ in range(14):
            mid = (lo + hi) >> 1
            probe = plsc.load_gather(ids_v, [mid])
            went_right = probe < lane
            lo = jnp.where(went_right, mid + 1, lo)
            hi = jnp.where(went_right, hi, mid)
        total = _N - lo

        cnt2_v[pl.ds(0, _LANES)] = total
        cnt2_v[pl.ds(_LANES, _LANES)] = jnp.zeros((_LANES,), jnp.int32)
        nxt = plsc.load_gather(cnt2_v, [lane + 1])
        n_vec = total - nxt

        lf_copy.wait()
        vals = plsc.load_gather(lf_v, [n_vec])
        tot = jnp.sum(vals)
        neg_g = jnp.sum((0 - n_vec) >> 31)
        idx = jnp.zeros((_LANES,), jnp.int32) + ((_RECIP_BASE - 1) - neg_g)
        recip = plsc.load_gather(lf_v, [idx])
        outf_v[...] = (jnp.zeros((_LANES,), jnp.float32) + tot) * recip
        pltpu.sync_copy(outf_v, loss_hbm)


def kernel(y_pred, y_true, group_ids):
    del y_pred, y_true
    loss = _pl_listwise_loss(group_ids, _LF_TABLE)
    return loss[0]
